# Initial kernel scaffold; baseline (speedup 1.0000x reference)
import functools

import jax
import jax.numpy as jnp
from jax import lax
from jax.experimental import pallas as pl
from jax.experimental.pallas import tpu as pltpu
from jax.experimental.pallas import tpu_sc as plsc

N = 10000
E = 320000
F = 128
NC = 2
NS = 16
L = 16

CHUNK = 128
EDGES_PER_CORE = E // NC
CHUNKS_PER_CORE = EDGES_PER_CORE // CHUNK
CHUNKS_PER_TILE = -(-CHUNKS_PER_CORE // NS)
ROWS_PER_TILE = N // NS
ZROWS = 125

_mesh = plsc.VectorSubcoreMesh(
    core_axis_name="c", subcore_axis_name="s", num_cores=NC, num_subcores=NS
)


def _spmm_sc(support, src, dst, vals):
    @functools.partial(
        pl.kernel,
        out_type=jax.ShapeDtypeStruct((NC * N, F), jnp.float32),
        mesh=_mesh,
        scratch_types=[
            pltpu.VMEM_SHARED((N, F), jnp.float32),
            pltpu.VMEM((CHUNK,), jnp.int32),
            pltpu.VMEM((CHUNK,), jnp.int32),
            pltpu.VMEM((CHUNK,), jnp.float32),
            pltpu.VMEM((CHUNK, F), jnp.float32),
            pltpu.VMEM((ZROWS, F), jnp.float32),
        ],
    )
    def k(sup_hbm, src_hbm, dst_hbm, vals_hbm, out_hbm,
          acc, srcv, dstv, valv, rows, zbuf):
        cid = lax.axis_index("c")
        sid = lax.axis_index("s")

        zvec = jnp.zeros((L,), jnp.float32)

        @pl.loop(0, ZROWS)
        def _(r):
            for j in range(F // L):
                zbuf[r, pl.ds(j * L, L)] = zvec

        @pl.loop(0, ROWS_PER_TILE // ZROWS)
        def _(i):
            pltpu.sync_copy(zbuf, acc.at[pl.ds(sid * ROWS_PER_TILE + i * ZROWS, ZROWS)])

        plsc.subcore_barrier()

        @pl.loop(0, CHUNKS_PER_TILE)
        def _(t):
            chunk = t * NS + sid

            @pl.when(chunk < CHUNKS_PER_CORE)
            def _():
                base = (cid * CHUNKS_PER_CORE + chunk) * CHUNK
                pltpu.sync_copy(src_hbm.at[pl.ds(base, CHUNK)], srcv)
                pltpu.sync_copy(dst_hbm.at[pl.ds(base, CHUNK)], dstv)
                pltpu.sync_copy(vals_hbm.at[pl.ds(base, CHUNK)], valv)
                pltpu.sync_copy(sup_hbm.at[srcv], rows)

                @pl.loop(0, CHUNK)
                def _(e):
                    v = valv[e]
                    for j in range(F // L):
                        sl = (e, pl.ds(j * L, L))
                        rows[sl] = rows[sl] * v

                pltpu.sync_copy(rows, acc.at[dstv], add=True)

        plsc.subcore_barrier()

        @pl.loop(0, ROWS_PER_TILE // ZROWS)
        def _(i):
            off = sid * ROWS_PER_TILE + i * ZROWS
            pltpu.sync_copy(acc.at[pl.ds(off, ZROWS)],
                            out_hbm.at[pl.ds(cid * N + off, ZROWS)])

    return k(support, src, dst, vals)


_BM = 1000


def _dot(a, b):
    return lax.dot_general(a, b, (((1,), (0,)), ((), ())),
                           precision=lax.Precision.HIGHEST,
                           preferred_element_type=jnp.float32)


def _tc_matmul(x, W):
    def body(x_ref, w_ref, o_ref):
        o_ref[...] = _dot(x_ref[...], w_ref[...])

    return pl.pallas_call(
        body,
        grid=(N // _BM,),
        in_specs=[pl.BlockSpec((_BM, F), lambda i: (i, 0)),
                  pl.BlockSpec((F, F), lambda i: (0, 0))],
        out_specs=pl.BlockSpec((_BM, F), lambda i: (i, 0)),
        out_shape=jax.ShapeDtypeStruct((N, F), jnp.float32),
    )(x, W)


def _tc_relu_matmul(p, b, W):
    def body(p0_ref, p1_ref, b_ref, w_ref, o_ref):
        h = jax.nn.relu(p0_ref[...] + p1_ref[...] + b_ref[...])
        o_ref[...] = _dot(h, w_ref[...])

    nb = N // _BM
    return pl.pallas_call(
        body,
        grid=(nb,),
        in_specs=[pl.BlockSpec((_BM, F), lambda i: (i, 0)),
                  pl.BlockSpec((_BM, F), lambda i, nb=nb: (i + nb, 0)),
                  pl.BlockSpec((1, F), lambda i: (0, 0)),
                  pl.BlockSpec((F, F), lambda i: (0, 0))],
        out_specs=pl.BlockSpec((_BM, F), lambda i: (i, 0)),
        out_shape=jax.ShapeDtypeStruct((N, F), jnp.float32),
    )(p, p, b.reshape(1, F), W)


def _tc_relu_bias(p, b):
    def body(p0_ref, p1_ref, b_ref, o_ref):
        o_ref[...] = jax.nn.relu(p0_ref[...] + p1_ref[...] + b_ref[...])

    nb = N // _BM
    return pl.pallas_call(
        body,
        grid=(nb,),
        in_specs=[pl.BlockSpec((_BM, F), lambda i: (i, 0)),
                  pl.BlockSpec((_BM, F), lambda i, nb=nb: (i + nb, 0)),
                  pl.BlockSpec((1, F), lambda i: (0, 0))],
        out_specs=pl.BlockSpec((_BM, F), lambda i: (i, 0)),
        out_shape=jax.ShapeDtypeStruct((N, F), jnp.float32),
    )(p, p, b.reshape(1, F))


def kernel(x, adj_indices, adj_values, W1, b1, W2, b2):
    dst = adj_indices[0]
    src = adj_indices[1]
    s1 = _tc_matmul(x, W1)
    p1 = _spmm_sc(s1, src, dst, adj_values)
    s2 = _tc_relu_matmul(p1, b1, W2)
    p2 = _spmm_sc(s2, src, dst, adj_values)
    return _tc_relu_bias(p2, b2)


# speedup vs baseline: 4.4395x; 4.4395x over previous
import dataclasses
import functools

import jax
import jax.numpy as jnp
from jax import lax
from jax.experimental import pallas as pl
from jax.experimental.pallas import tpu as pltpu
from jax.experimental.pallas import tpu_sc as plsc

N = 10000
E = 320000
F = 128
NC = 2
NS = 16
L = 16

CHUNK = 128
EDGES_PER_CORE = E // NC
CHUNKS_PER_CORE = EDGES_PER_CORE // CHUNK
CHUNKS_PER_TILE = -(-CHUNKS_PER_CORE // NS)
NPAD = 10240
ROWS_PER_TILE = NPAD // NS
ZROWS = 128

_mesh = plsc.VectorSubcoreMesh(
    core_axis_name="c", subcore_axis_name="s", num_cores=NC, num_subcores=NS
)

_sc_params = pltpu.CompilerParams()
if "needs_layout_passes" in pltpu.CompilerParams.__dataclass_fields__:
    _sc_params = dataclasses.replace(_sc_params, needs_layout_passes=False)


def _spmm_sc(support, src, dst, vals):
    @functools.partial(
        pl.kernel,
        out_type=(jax.ShapeDtypeStruct((NPAD, F), jnp.float32),
                  jax.ShapeDtypeStruct((NPAD, F), jnp.float32)),
        mesh=_mesh,
        compiler_params=_sc_params,
        scratch_types=[
            pltpu.VMEM_SHARED((NPAD, F), jnp.float32),
            pltpu.VMEM((CHUNK,), jnp.int32),
            pltpu.VMEM((CHUNK,), jnp.int32),
            pltpu.VMEM((CHUNK,), jnp.float32),
            pltpu.VMEM((CHUNK, F), jnp.float32),
            pltpu.VMEM((ZROWS, F), jnp.float32),
        ],
    )
    def k(sup_hbm, src_hbm, dst_hbm, vals_hbm, outa_hbm, outb_hbm,
          acc, srcv, dstv, valv, rows, zbuf):
        cid = lax.axis_index("c")
        sid = lax.axis_index("s")

        zvec = jnp.zeros((L,), jnp.float32)

        @pl.loop(0, ZROWS)
        def _(r):
            for j in range(F // L):
                zbuf[r, pl.ds(j * L, L)] = zvec

        @pl.loop(0, ROWS_PER_TILE // ZROWS)
        def _(i):
            pltpu.sync_copy(zbuf, acc.at[pl.ds(sid * ROWS_PER_TILE + i * ZROWS, ZROWS)])

        plsc.subcore_barrier()

        @pl.loop(0, CHUNKS_PER_TILE)
        def _(t):
            chunk = t * NS + sid

            @pl.when(chunk < CHUNKS_PER_CORE)
            def _():
                base = (cid * CHUNKS_PER_CORE + chunk) * CHUNK
                pltpu.sync_copy(src_hbm.at[pl.ds(base, CHUNK)], srcv)
                pltpu.sync_copy(dst_hbm.at[pl.ds(base, CHUNK)], dstv)
                pltpu.sync_copy(vals_hbm.at[pl.ds(base, CHUNK)], valv)
                pltpu.sync_copy(sup_hbm.at[srcv], rows)

                @pl.loop(0, CHUNK)
                def _(e):
                    v = plsc.load_gather(valv, [jnp.full((L,), e, jnp.int32)])
                    for j in range(F // L):
                        sl = (e, pl.ds(j * L, L))
                        rows[sl] = rows[sl] * v

                pltpu.sync_copy(rows, acc.at[dstv], add=True)

        plsc.subcore_barrier()

        @pl.loop(0, ROWS_PER_TILE // ZROWS)
        def _(i):
            off = sid * ROWS_PER_TILE + i * ZROWS

            @pl.when(cid == 0)
            def _():
                pltpu.sync_copy(acc.at[pl.ds(off, ZROWS)],
                                outa_hbm.at[pl.ds(off, ZROWS)])

            @pl.when(cid == 1)
            def _():
                pltpu.sync_copy(acc.at[pl.ds(off, ZROWS)],
                                outb_hbm.at[pl.ds(off, ZROWS)])

    return k(support, src, dst, vals)


_BM = 1000


def _dot(a, b):
    return lax.dot_general(a, b, (((1,), (0,)), ((), ())),
                           precision=lax.Precision.HIGHEST,
                           preferred_element_type=jnp.float32)


def _tc_matmul(x, W):
    def body(x_ref, w_ref, o_ref):
        o_ref[...] = _dot(x_ref[...], w_ref[...])

    return pl.pallas_call(
        body,
        grid=(N // _BM,),
        in_specs=[pl.BlockSpec((_BM, F), lambda i: (i, 0)),
                  pl.BlockSpec((F, F), lambda i: (0, 0))],
        out_specs=pl.BlockSpec((_BM, F), lambda i: (i, 0)),
        out_shape=jax.ShapeDtypeStruct((N, F), jnp.float32),
    )(x, W)


def _tc_relu_matmul(pa, pb, b, W):
    def body(p0_ref, p1_ref, b_ref, w_ref, o_ref):
        h = jax.nn.relu(p0_ref[...] + p1_ref[...] + b_ref[...])
        o_ref[...] = _dot(h, w_ref[...])

    return pl.pallas_call(
        body,
        grid=(N // _BM,),
        in_specs=[pl.BlockSpec((_BM, F), lambda i: (i, 0)),
                  pl.BlockSpec((_BM, F), lambda i: (i, 0)),
                  pl.BlockSpec((1, F), lambda i: (0, 0)),
                  pl.BlockSpec((F, F), lambda i: (0, 0))],
        out_specs=pl.BlockSpec((_BM, F), lambda i: (i, 0)),
        out_shape=jax.ShapeDtypeStruct((N, F), jnp.float32),
    )(pa, pb, b.reshape(1, F), W)


def _tc_relu_bias(pa, pb, b):
    def body(p0_ref, p1_ref, b_ref, o_ref):
        o_ref[...] = jax.nn.relu(p0_ref[...] + p1_ref[...] + b_ref[...])

    return pl.pallas_call(
        body,
        grid=(N // _BM,),
        in_specs=[pl.BlockSpec((_BM, F), lambda i: (i, 0)),
                  pl.BlockSpec((_BM, F), lambda i: (i, 0)),
                  pl.BlockSpec((1, F), lambda i: (0, 0))],
        out_specs=pl.BlockSpec((_BM, F), lambda i: (i, 0)),
        out_shape=jax.ShapeDtypeStruct((N, F), jnp.float32),
    )(pa, pb, b.reshape(1, F))


def kernel(x, adj_indices, adj_values, W1, b1, W2, b2):
    dst = adj_indices[0]
    src = adj_indices[1]
    s1 = _tc_matmul(x, W1)
    p1a, p1b = _spmm_sc(s1, src, dst, adj_values)
    s2 = _tc_relu_matmul(p1a, p1b, b1, W2)
    p2a, p2b = _spmm_sc(s2, src, dst, adj_values)
    return _tc_relu_bias(p2a, p2b, b2)


# speedup vs baseline: 7.5017x; 1.6898x over previous
import dataclasses
import functools

import jax
import jax.numpy as jnp
from jax import lax
from jax.experimental import pallas as pl
from jax.experimental.pallas import tpu as pltpu
from jax.experimental.pallas import tpu_sc as plsc

N = 10000
E = 320000
F = 128
NC = 2
NS = 16
L = 16

CHUNK = 128
CPT = 80
CHUNKS_PER_CORE = CPT * NS
NCHUNKS = CHUNKS_PER_CORE * NC
NARR = NCHUNKS + 2 * NS
EARR = NARR * CHUNK
NPAD = 10240
ROWS_PER_TILE = NPAD // NS
ZROWS = 128

_mesh = plsc.VectorSubcoreMesh(
    core_axis_name="c", subcore_axis_name="s", num_cores=NC, num_subcores=NS
)

_sc_params = pltpu.CompilerParams()
if "needs_layout_passes" in pltpu.CompilerParams.__dataclass_fields__:
    _sc_params = dataclasses.replace(_sc_params, needs_layout_passes=False)


def _spmm_sc(support, src, dst, vals):
    @functools.partial(
        pl.kernel,
        out_type=(jax.ShapeDtypeStruct((NPAD, F), jnp.float32),
                  jax.ShapeDtypeStruct((NPAD, F), jnp.float32)),
        mesh=_mesh,
        compiler_params=_sc_params,
        scratch_types=[
            pltpu.VMEM_SHARED((NPAD, F), jnp.float32),
            pltpu.VMEM((2, CHUNK), jnp.int32),
            pltpu.VMEM((2, 1, CHUNK), jnp.int32),
            pltpu.VMEM((2, CHUNK), jnp.float32),
            pltpu.VMEM((2, CHUNK, F), jnp.float32),
            pltpu.SemaphoreType.DMA,
            pltpu.SemaphoreType.DMA,
            pltpu.SemaphoreType.DMA,
            pltpu.SemaphoreType.DMA,
        ],
    )
    def k(sup_hbm, src_hbm, dst_hbm, vals_hbm, outa_hbm, outb_hbm,
          acc, srcv, dstv, valv, rows, si0, si1, sg0, sg1):
        cid = lax.axis_index("c")
        sid = lax.axis_index("s")
        sis = (si0, si1)
        sgs = (sg0, sg1)

        def chunk_base(ti):
            return (cid * CHUNKS_PER_CORE + ti * NS + sid) * CHUNK

        def issue_idx_fetch(ti, b):
            base = chunk_base(ti)
            sem = sis[b]
            pltpu.async_copy(src_hbm.at[pl.ds(base, CHUNK)], srcv.at[b], sem)
            pltpu.async_copy(dst_hbm.at[pl.ds(base, CHUNK)], dstv.at[b, 0], sem)
            pltpu.async_copy(vals_hbm.at[pl.ds(base, CHUNK)], valv.at[b], sem)

        def wait_idx_fetch(b):
            sem = sis[b]
            pltpu.make_async_copy(src_hbm.at[pl.ds(0, CHUNK)], srcv.at[b], sem).wait()
            pltpu.make_async_copy(dst_hbm.at[pl.ds(0, CHUNK)], dstv.at[b, 0], sem).wait()
            pltpu.make_async_copy(vals_hbm.at[pl.ds(0, CHUNK)], valv.at[b], sem).wait()

        def issue_gather(b):
            pltpu.async_copy(sup_hbm.at[srcv.at[b]], rows.at[b], sgs[b])

        def wait_gather(b):
            pltpu.make_async_copy(sup_hbm.at[srcv.at[b]], rows.at[b], sgs[b]).wait()

        zvec = jnp.zeros((L,), jnp.float32)

        @pl.loop(0, ZROWS)
        def _(r):
            for j in range(F // L):
                rows[0, r, pl.ds(j * L, L)] = zvec

        @pl.loop(0, ROWS_PER_TILE // ZROWS)
        def _(i):
            pltpu.sync_copy(rows.at[0],
                            acc.at[pl.ds(sid * ROWS_PER_TILE + i * ZROWS, ZROWS)])

        plsc.subcore_barrier()

        issue_idx_fetch(0, 0)
        issue_idx_fetch(1, 1)
        wait_idx_fetch(0)
        issue_gather(0)

        @pl.loop(0, CPT, step=2)
        def _(t):
            for b in range(2):
                ti = t + b
                nb = 1 - b
                wait_idx_fetch(nb)
                issue_gather(nb)
                wait_gather(b)

                @pl.loop(0, CHUNK)
                def _(e):
                    v = plsc.load_gather(
                        valv, [jnp.full((L,), b, jnp.int32),
                               jnp.full((L,), e, jnp.int32)])
                    for j in range(F // L):
                        sl = (b, e, pl.ds(j * L, L))
                        rows[sl] = rows[sl] * v

                pltpu.sync_copy(rows.at[b], acc.at[dstv.at[b, 0]], add=True)

                issue_idx_fetch(ti + 2, b)

        wait_gather(0)
        wait_idx_fetch(1)

        plsc.subcore_barrier()

        @pl.loop(0, ROWS_PER_TILE // ZROWS)
        def _(i):
            off = sid * ROWS_PER_TILE + i * ZROWS

            @pl.when(cid == 0)
            def _():
                pltpu.sync_copy(acc.at[pl.ds(off, ZROWS)],
                                outa_hbm.at[pl.ds(off, ZROWS)])

            @pl.when(cid == 1)
            def _():
                pltpu.sync_copy(acc.at[pl.ds(off, ZROWS)],
                                outb_hbm.at[pl.ds(off, ZROWS)])

    return k(support, src, dst, vals)


_BM = 1000


def _dot(a, b):
    return lax.dot_general(a, b, (((1,), (0,)), ((), ())),
                           precision=lax.Precision.HIGHEST,
                           preferred_element_type=jnp.float32)


def _tc_matmul(x, W):
    def body(x_ref, w_ref, o_ref):
        o_ref[...] = _dot(x_ref[...], w_ref[...])

    return pl.pallas_call(
        body,
        grid=(N // _BM,),
        in_specs=[pl.BlockSpec((_BM, F), lambda i: (i, 0)),
                  pl.BlockSpec((F, F), lambda i: (0, 0))],
        out_specs=pl.BlockSpec((_BM, F), lambda i: (i, 0)),
        out_shape=jax.ShapeDtypeStruct((N, F), jnp.float32),
    )(x, W)


def _tc_relu_matmul(pa, pb, b, W):
    def body(p0_ref, p1_ref, b_ref, w_ref, o_ref):
        h = jax.nn.relu(p0_ref[...] + p1_ref[...] + b_ref[...])
        o_ref[...] = _dot(h, w_ref[...])

    return pl.pallas_call(
        body,
        grid=(N // _BM,),
        in_specs=[pl.BlockSpec((_BM, F), lambda i: (i, 0)),
                  pl.BlockSpec((_BM, F), lambda i: (i, 0)),
                  pl.BlockSpec((1, F), lambda i: (0, 0)),
                  pl.BlockSpec((F, F), lambda i: (0, 0))],
        out_specs=pl.BlockSpec((_BM, F), lambda i: (i, 0)),
        out_shape=jax.ShapeDtypeStruct((N, F), jnp.float32),
    )(pa, pb, b.reshape(1, F), W)


def _tc_relu_bias(pa, pb, b):
    def body(p0_ref, p1_ref, b_ref, o_ref):
        o_ref[...] = jax.nn.relu(p0_ref[...] + p1_ref[...] + b_ref[...])

    return pl.pallas_call(
        body,
        grid=(N // _BM,),
        in_specs=[pl.BlockSpec((_BM, F), lambda i: (i, 0)),
                  pl.BlockSpec((_BM, F), lambda i: (i, 0)),
                  pl.BlockSpec((1, F), lambda i: (0, 0))],
        out_specs=pl.BlockSpec((_BM, F), lambda i: (i, 0)),
        out_shape=jax.ShapeDtypeStruct((N, F), jnp.float32),
    )(pa, pb, b.reshape(1, F))


def kernel(x, adj_indices, adj_values, W1, b1, W2, b2):
    dst = adj_indices[0]
    src = adj_indices[1]
    pad = EARR - E
    pidx = jnp.arange(pad, dtype=jnp.int32) % N
    src_p = jnp.concatenate([src, pidx])
    dst_p = jnp.concatenate([dst, pidx])
    vals_p = jnp.concatenate([adj_values, jnp.zeros((pad,), jnp.float32)])
    s1 = _tc_matmul(x, W1)
    p1a, p1b = _spmm_sc(s1, src_p, dst_p, vals_p)
    s2 = _tc_relu_matmul(p1a, p1b, b1, W2)
    p2a, p2b = _spmm_sc(s2, src_p, dst_p, vals_p)
    return _tc_relu_bias(p2a, p2b, b2)


# speedup vs baseline: 7.7300x; 1.0304x over previous
import dataclasses
import functools

import jax
import jax.numpy as jnp
from jax import lax
from jax.experimental import pallas as pl
from jax.experimental.pallas import tpu as pltpu
from jax.experimental.pallas import tpu_sc as plsc

N = 10000
E = 320000
F = 128
NC = 2
NS = 16
L = 16

CHUNK = 128
CPT = 80
CHUNKS_PER_CORE = CPT * NS
NCHUNKS = CHUNKS_PER_CORE * NC
NARR = NCHUNKS + 2 * NS
EARR = NARR * CHUNK
NPAD = 10240
ROWS_PER_TILE = NPAD // NS
ZROWS = 128

_mesh = plsc.VectorSubcoreMesh(
    core_axis_name="c", subcore_axis_name="s", num_cores=NC, num_subcores=NS
)

_sc_params = pltpu.CompilerParams()
if "needs_layout_passes" in pltpu.CompilerParams.__dataclass_fields__:
    _sc_params = dataclasses.replace(_sc_params, needs_layout_passes=False)


def _spmm_sc(support, src, dst, vals):
    @functools.partial(
        pl.kernel,
        out_type=(jax.ShapeDtypeStruct((NPAD, F), jnp.float32),
                  jax.ShapeDtypeStruct((NPAD, F), jnp.float32)),
        mesh=_mesh,
        compiler_params=_sc_params,
        scratch_types=[
            pltpu.VMEM_SHARED((NPAD, F), jnp.float32),
            pltpu.VMEM((2, CHUNK), jnp.int32),
            pltpu.VMEM((2, 1, CHUNK), jnp.int32),
            pltpu.VMEM((2, CHUNK), jnp.float32),
            pltpu.VMEM((2, CHUNK, F), jnp.float32),
            pltpu.SemaphoreType.DMA,
            pltpu.SemaphoreType.DMA,
            pltpu.SemaphoreType.DMA,
            pltpu.SemaphoreType.DMA,
        ],
    )
    def k(sup_hbm, src_hbm, dst_hbm, vals_hbm, outa_hbm, outb_hbm,
          acc, srcv, dstv, valv, rows, si0, si1, sg0, sg1):
        cid = lax.axis_index("c")
        sid = lax.axis_index("s")
        sis = (si0, si1)
        sgs = (sg0, sg1)

        def chunk_base(ti):
            return (cid * CHUNKS_PER_CORE + ti * NS + sid) * CHUNK

        def issue_idx_fetch(ti, b):
            base = chunk_base(ti)
            sem = sis[b]
            pltpu.async_copy(src_hbm.at[pl.ds(base, CHUNK)], srcv.at[b], sem)
            pltpu.async_copy(dst_hbm.at[pl.ds(base, CHUNK)], dstv.at[b, 0], sem)
            pltpu.async_copy(vals_hbm.at[pl.ds(base, CHUNK)], valv.at[b], sem)

        def wait_idx_fetch(b):
            sem = sis[b]
            pltpu.make_async_copy(src_hbm.at[pl.ds(0, CHUNK)], srcv.at[b], sem).wait()
            pltpu.make_async_copy(dst_hbm.at[pl.ds(0, CHUNK)], dstv.at[b, 0], sem).wait()
            pltpu.make_async_copy(vals_hbm.at[pl.ds(0, CHUNK)], valv.at[b], sem).wait()

        def issue_gather(b):
            pltpu.async_copy(sup_hbm.at[srcv.at[b]], rows.at[b], sgs[b])

        def wait_gather(b):
            pltpu.make_async_copy(sup_hbm.at[srcv.at[b]], rows.at[b], sgs[b]).wait()

        zvec = jnp.zeros((L,), jnp.float32)

        @pl.loop(0, ZROWS)
        def _(r):
            for j in range(F // L):
                rows[0, r, pl.ds(j * L, L)] = zvec

        @pl.loop(0, ROWS_PER_TILE // ZROWS)
        def _(i):
            pltpu.sync_copy(rows.at[0],
                            acc.at[pl.ds(sid * ROWS_PER_TILE + i * ZROWS, ZROWS)])

        plsc.subcore_barrier()

        issue_idx_fetch(0, 0)
        issue_idx_fetch(1, 1)
        wait_idx_fetch(0)
        issue_gather(0)

        @pl.loop(0, CPT, step=2)
        def _(t):
            for b in range(2):
                ti = t + b
                nb = 1 - b
                wait_idx_fetch(nb)
                issue_gather(nb)
                wait_gather(b)

                @pl.loop(0, CHUNK, unroll=4)
                def _(e):
                    v = plsc.load_gather(
                        valv, [jnp.full((L,), b, jnp.int32),
                               jnp.full((L,), e, jnp.int32)])
                    for j in range(F // L):
                        sl = (b, e, pl.ds(j * L, L))
                        rows[sl] = rows[sl] * v

                pltpu.sync_copy(rows.at[b], acc.at[dstv.at[b, 0]], add=True)

                issue_idx_fetch(ti + 2, b)

        wait_gather(0)
        wait_idx_fetch(1)

        plsc.subcore_barrier()

        @pl.loop(0, ROWS_PER_TILE // ZROWS)
        def _(i):
            off = sid * ROWS_PER_TILE + i * ZROWS

            @pl.when(cid == 0)
            def _():
                pltpu.sync_copy(acc.at[pl.ds(off, ZROWS)],
                                outa_hbm.at[pl.ds(off, ZROWS)])

            @pl.when(cid == 1)
            def _():
                pltpu.sync_copy(acc.at[pl.ds(off, ZROWS)],
                                outb_hbm.at[pl.ds(off, ZROWS)])

    return k(support, src, dst, vals)


_BM = 1000


def _dot(a, b):
    return lax.dot_general(a, b, (((1,), (0,)), ((), ())),
                           precision=lax.Precision.HIGHEST,
                           preferred_element_type=jnp.float32)


def _tc_matmul(x, W):
    def body(x_ref, w_ref, o_ref):
        o_ref[...] = _dot(x_ref[...], w_ref[...])

    return pl.pallas_call(
        body,
        grid=(N // _BM,),
        in_specs=[pl.BlockSpec((_BM, F), lambda i: (i, 0)),
                  pl.BlockSpec((F, F), lambda i: (0, 0))],
        out_specs=pl.BlockSpec((_BM, F), lambda i: (i, 0)),
        out_shape=jax.ShapeDtypeStruct((N, F), jnp.float32),
    )(x, W)


def _tc_relu_matmul(pa, pb, b, W):
    def body(p0_ref, p1_ref, b_ref, w_ref, o_ref):
        h = jax.nn.relu(p0_ref[...] + p1_ref[...] + b_ref[...])
        o_ref[...] = _dot(h, w_ref[...])

    return pl.pallas_call(
        body,
        grid=(N // _BM,),
        in_specs=[pl.BlockSpec((_BM, F), lambda i: (i, 0)),
                  pl.BlockSpec((_BM, F), lambda i: (i, 0)),
                  pl.BlockSpec((1, F), lambda i: (0, 0)),
                  pl.BlockSpec((F, F), lambda i: (0, 0))],
        out_specs=pl.BlockSpec((_BM, F), lambda i: (i, 0)),
        out_shape=jax.ShapeDtypeStruct((N, F), jnp.float32),
    )(pa, pb, b.reshape(1, F), W)


def _tc_relu_bias(pa, pb, b):
    def body(p0_ref, p1_ref, b_ref, o_ref):
        o_ref[...] = jax.nn.relu(p0_ref[...] + p1_ref[...] + b_ref[...])

    return pl.pallas_call(
        body,
        grid=(N // _BM,),
        in_specs=[pl.BlockSpec((_BM, F), lambda i: (i, 0)),
                  pl.BlockSpec((_BM, F), lambda i: (i, 0)),
                  pl.BlockSpec((1, F), lambda i: (0, 0))],
        out_specs=pl.BlockSpec((_BM, F), lambda i: (i, 0)),
        out_shape=jax.ShapeDtypeStruct((N, F), jnp.float32),
    )(pa, pb, b.reshape(1, F))


def kernel(x, adj_indices, adj_values, W1, b1, W2, b2):
    dst = adj_indices[0]
    src = adj_indices[1]
    pad = EARR - E
    pidx = jnp.arange(pad, dtype=jnp.int32) % N
    src_p = jnp.concatenate([src, pidx])
    dst_p = jnp.concatenate([dst, pidx])
    vals_p = jnp.concatenate([adj_values, jnp.zeros((pad,), jnp.float32)])
    s1 = _tc_matmul(x, W1)
    p1a, p1b = _spmm_sc(s1, src_p, dst_p, vals_p)
    s2 = _tc_relu_matmul(p1a, p1b, b1, W2)
    p2a, p2b = _spmm_sc(s2, src_p, dst_p, vals_p)
    return _tc_relu_bias(p2a, p2b, b2)


# speedup vs baseline: 9.3789x; 1.2133x over previous
import dataclasses
import functools

import jax
import jax.numpy as jnp
from jax import lax
from jax.experimental import pallas as pl
from jax.experimental.pallas import tpu as pltpu
from jax.experimental.pallas import tpu_sc as plsc

N = 10000
E = 320000
F = 128
NC = 2
NS = 16
L = 16

CHUNK = 128
CPT = 80
CHUNKS_PER_CORE = CPT * NS
NCHUNKS = CHUNKS_PER_CORE * NC
NARR = NCHUNKS + 2 * NS
EARR = NARR * CHUNK
NPAD = 10240
ROWS_PER_TILE = NPAD // NS
ZROWS = 128

_mesh = plsc.VectorSubcoreMesh(
    core_axis_name="c", subcore_axis_name="s", num_cores=NC, num_subcores=NS
)

_sc_params = pltpu.CompilerParams()
if "needs_layout_passes" in pltpu.CompilerParams.__dataclass_fields__:
    _sc_params = dataclasses.replace(_sc_params, needs_layout_passes=False)


def _spmm_sc(support, src, dst, vals):
    @functools.partial(
        pl.kernel,
        out_type=(jax.ShapeDtypeStruct((NPAD, F), jnp.float32),
                  jax.ShapeDtypeStruct((NPAD, F), jnp.float32)),
        mesh=_mesh,
        compiler_params=_sc_params,
        scratch_types=[
            pltpu.VMEM_SHARED((NPAD, F), jnp.float32),
            pltpu.VMEM((2, CHUNK), jnp.int32),
            pltpu.VMEM((2, 1, CHUNK), jnp.int32),
            pltpu.VMEM((2, CHUNK), jnp.float32),
            pltpu.VMEM((2, CHUNK, F), jnp.float32),
            pltpu.SemaphoreType.DMA,
            pltpu.SemaphoreType.DMA,
            pltpu.SemaphoreType.DMA,
            pltpu.SemaphoreType.DMA,
            pltpu.SemaphoreType.DMA,
            pltpu.SemaphoreType.DMA,
            pltpu.SemaphoreType.DMA,
            pltpu.SemaphoreType.DMA,
        ],
    )
    def k(sup_hbm, src_hbm, dst_hbm, vals_hbm, outa_hbm, outb_hbm,
          acc, srcv, dstv, valv, rows, si0, si1, sg0, sg1, sd0, sd1, ss0, ss1):
        cid = lax.axis_index("c")
        sid = lax.axis_index("s")
        sis = (si0, si1)
        sgs = (sg0, sg1)
        sds = (sd0, sd1)
        sss = (ss0, ss1)

        def chunk_base(ti):
            return (cid * CHUNKS_PER_CORE + ti * NS + sid) * CHUNK

        def issue_srcval_fetch(ti, b):
            base = chunk_base(ti)
            sem = sis[b]
            pltpu.async_copy(src_hbm.at[pl.ds(base, CHUNK)], srcv.at[b], sem)
            pltpu.async_copy(vals_hbm.at[pl.ds(base, CHUNK)], valv.at[b], sem)

        def wait_srcval_fetch(b):
            sem = sis[b]
            pltpu.make_async_copy(src_hbm.at[pl.ds(0, CHUNK)], srcv.at[b], sem).wait()
            pltpu.make_async_copy(vals_hbm.at[pl.ds(0, CHUNK)], valv.at[b], sem).wait()

        def issue_dst_fetch(ti, b):
            pltpu.async_copy(dst_hbm.at[pl.ds(chunk_base(ti), CHUNK)],
                             dstv.at[b, 0], sds[b])

        def wait_dst_fetch(b):
            pltpu.make_async_copy(dst_hbm.at[pl.ds(0, CHUNK)],
                                  dstv.at[b, 0], sds[b]).wait()

        def issue_gather(b):
            pltpu.async_copy(sup_hbm.at[srcv.at[b]], rows.at[b], sgs[b])

        def wait_gather(b):
            pltpu.make_async_copy(sup_hbm.at[srcv.at[b]], rows.at[b], sgs[b]).wait()

        def issue_scatter(b):
            pltpu.async_copy(rows.at[b], acc.at[dstv.at[b, 0]], sss[b], add=True)

        def wait_scatter(b):
            pltpu.make_async_copy(rows.at[b], acc.at[dstv.at[b, 0]], sss[b]).wait()

        zvec = jnp.zeros((L,), jnp.float32)

        @pl.loop(0, ZROWS)
        def _(r):
            for j in range(F // L):
                rows[0, r, pl.ds(j * L, L)] = zvec

        @pl.loop(0, ROWS_PER_TILE // ZROWS)
        def _(i):
            pltpu.sync_copy(rows.at[0],
                            acc.at[pl.ds(sid * ROWS_PER_TILE + i * ZROWS, ZROWS)])

        plsc.subcore_barrier()

        def scale(b):
            @pl.loop(0, CHUNK, unroll=4)
            def _(e):
                v = plsc.load_gather(
                    valv, [jnp.full((L,), b, jnp.int32),
                           jnp.full((L,), e, jnp.int32)])
                for j in range(F // L):
                    sl = (b, e, pl.ds(j * L, L))
                    rows[sl] = rows[sl] * v

        def half(ti, b, first=False):
            nb = 1 - b
            wait_srcval_fetch(nb)
            if not first:
                wait_scatter(nb)
            issue_dst_fetch(ti + 1, nb)
            issue_gather(nb)
            wait_gather(b)
            scale(b)
            wait_dst_fetch(b)
            issue_scatter(b)
            issue_srcval_fetch(ti + 2, b)

        issue_srcval_fetch(0, 0)
        issue_srcval_fetch(1, 1)
        wait_srcval_fetch(0)
        issue_dst_fetch(0, 0)
        issue_gather(0)

        half(0, 0, first=True)

        @pl.loop(0, (CPT - 2) // 2)
        def _(t):
            ti = 1 + 2 * t
            half(ti, 1)
            half(ti + 1, 0)

        half(CPT - 1, 1)

        wait_scatter(1)
        wait_gather(0)
        wait_dst_fetch(0)
        wait_srcval_fetch(1)

        plsc.subcore_barrier()

        @pl.loop(0, ROWS_PER_TILE // ZROWS)
        def _(i):
            off = sid * ROWS_PER_TILE + i * ZROWS

            @pl.when(cid == 0)
            def _():
                pltpu.sync_copy(acc.at[pl.ds(off, ZROWS)],
                                outa_hbm.at[pl.ds(off, ZROWS)])

            @pl.when(cid == 1)
            def _():
                pltpu.sync_copy(acc.at[pl.ds(off, ZROWS)],
                                outb_hbm.at[pl.ds(off, ZROWS)])

    return k(support, src, dst, vals)


_BM = 1000


def _dot(a, b):
    return lax.dot_general(a, b, (((1,), (0,)), ((), ())),
                           precision=lax.Precision.HIGHEST,
                           preferred_element_type=jnp.float32)


def _tc_matmul(x, W):
    def body(x_ref, w_ref, o_ref):
        o_ref[...] = _dot(x_ref[...], w_ref[...])

    return pl.pallas_call(
        body,
        grid=(N // _BM,),
        in_specs=[pl.BlockSpec((_BM, F), lambda i: (i, 0)),
                  pl.BlockSpec((F, F), lambda i: (0, 0))],
        out_specs=pl.BlockSpec((_BM, F), lambda i: (i, 0)),
        out_shape=jax.ShapeDtypeStruct((N, F), jnp.float32),
    )(x, W)


def _tc_relu_matmul(pa, pb, b, W):
    def body(p0_ref, p1_ref, b_ref, w_ref, o_ref):
        h = jax.nn.relu(p0_ref[...] + p1_ref[...] + b_ref[...])
        o_ref[...] = _dot(h, w_ref[...])

    return pl.pallas_call(
        body,
        grid=(N // _BM,),
        in_specs=[pl.BlockSpec((_BM, F), lambda i: (i, 0)),
                  pl.BlockSpec((_BM, F), lambda i: (i, 0)),
                  pl.BlockSpec((1, F), lambda i: (0, 0)),
                  pl.BlockSpec((F, F), lambda i: (0, 0))],
        out_specs=pl.BlockSpec((_BM, F), lambda i: (i, 0)),
        out_shape=jax.ShapeDtypeStruct((N, F), jnp.float32),
    )(pa, pb, b.reshape(1, F), W)


def _tc_relu_bias(pa, pb, b):
    def body(p0_ref, p1_ref, b_ref, o_ref):
        o_ref[...] = jax.nn.relu(p0_ref[...] + p1_ref[...] + b_ref[...])

    return pl.pallas_call(
        body,
        grid=(N // _BM,),
        in_specs=[pl.BlockSpec((_BM, F), lambda i: (i, 0)),
                  pl.BlockSpec((_BM, F), lambda i: (i, 0)),
                  pl.BlockSpec((1, F), lambda i: (0, 0))],
        out_specs=pl.BlockSpec((_BM, F), lambda i: (i, 0)),
        out_shape=jax.ShapeDtypeStruct((N, F), jnp.float32),
    )(pa, pb, b.reshape(1, F))


def kernel(x, adj_indices, adj_values, W1, b1, W2, b2):
    dst = adj_indices[0]
    src = adj_indices[1]
    pad = EARR - E
    pidx = jnp.arange(pad, dtype=jnp.int32) % N
    src_p = jnp.concatenate([src, pidx])
    dst_p = jnp.concatenate([dst, pidx])
    vals_p = jnp.concatenate([adj_values, jnp.zeros((pad,), jnp.float32)])
    s1 = _tc_matmul(x, W1)
    p1a, p1b = _spmm_sc(s1, src_p, dst_p, vals_p)
    s2 = _tc_relu_matmul(p1a, p1b, b1, W2)
    p2a, p2b = _spmm_sc(s2, src_p, dst_p, vals_p)
    return _tc_relu_bias(p2a, p2b, b2)


# speedup vs baseline: 11.0982x; 1.1833x over previous
import dataclasses
import functools

import jax
import jax.numpy as jnp
from jax import lax
from jax.experimental import pallas as pl
from jax.experimental.pallas import tpu as pltpu
from jax.experimental.pallas import tpu_sc as plsc

N = 10000
E = 320000
F = 128
NC = 2
NS = 16
L = 16

CHUNK = 128
CPT = 80
CHUNKS_PER_CORE = CPT * NS
NCHUNKS = CHUNKS_PER_CORE * NC
NARR = NCHUNKS + 2 * NS
EARR = NARR * CHUNK
NPAD = 10240
ROWS_PER_TILE = NPAD // NS
ZROWS = 128

_mesh = plsc.VectorSubcoreMesh(
    core_axis_name="c", subcore_axis_name="s", num_cores=NC, num_subcores=NS
)

_sc_params = pltpu.CompilerParams()
if "needs_layout_passes" in pltpu.CompilerParams.__dataclass_fields__:
    _sc_params = dataclasses.replace(_sc_params, needs_layout_passes=False)


def _spmm_sc(support, srcval, dst):
    @functools.partial(
        pl.kernel,
        out_type=(jax.ShapeDtypeStruct((NPAD, F), jnp.float32),
                  jax.ShapeDtypeStruct((NPAD, F), jnp.float32)),
        mesh=_mesh,
        compiler_params=_sc_params,
        scratch_types=[
            pltpu.VMEM_SHARED((NPAD, F), jnp.float32),
            pltpu.VMEM((2, 2 * CHUNK), jnp.int32),
            pltpu.VMEM((2, 1, CHUNK), jnp.int32),
            pltpu.VMEM((2, CHUNK, F), jnp.float32),
            pltpu.SemaphoreType.DMA,
            pltpu.SemaphoreType.DMA,
            pltpu.SemaphoreType.DMA,
            pltpu.SemaphoreType.DMA,
            pltpu.SemaphoreType.DMA,
            pltpu.SemaphoreType.DMA,
            pltpu.SemaphoreType.DMA,
            pltpu.SemaphoreType.DMA,
        ],
    )
    def k(sup_hbm, srcval_hbm, dst_hbm, outa_hbm, outb_hbm,
          acc, srcv, dstv, rows, si0, si1, sg0, sg1, sd0, sd1, ss0, ss1):
        cid = lax.axis_index("c")
        sid = lax.axis_index("s")
        sis = (si0, si1)
        sgs = (sg0, sg1)
        sds = (sd0, sd1)
        sss = (ss0, ss1)

        def chunk_base(ti):
            return (cid * CHUNKS_PER_CORE + ti * NS + sid) * CHUNK

        def issue_srcval_fetch(ti, b):
            pltpu.async_copy(srcval_hbm.at[pl.ds(chunk_base(ti) * 2, 2 * CHUNK)],
                             srcv.at[b], sis[b])

        def wait_srcval_fetch(b):
            pltpu.make_async_copy(srcval_hbm.at[pl.ds(0, 2 * CHUNK)],
                                  srcv.at[b], sis[b]).wait()

        def issue_dst_fetch(ti, b):
            pltpu.async_copy(dst_hbm.at[pl.ds(chunk_base(ti), CHUNK)],
                             dstv.at[b, 0], sds[b])

        def wait_dst_fetch(b):
            pltpu.make_async_copy(dst_hbm.at[pl.ds(0, CHUNK)],
                                  dstv.at[b, 0], sds[b]).wait()

        def issue_gather(b):
            pltpu.async_copy(sup_hbm.at[srcv.at[b, pl.ds(0, CHUNK)]],
                             rows.at[b], sgs[b])

        def wait_gather(b):
            pltpu.make_async_copy(sup_hbm.at[srcv.at[b, pl.ds(0, CHUNK)]],
                                  rows.at[b], sgs[b]).wait()

        def issue_scatter(b):
            pltpu.async_copy(rows.at[b], acc.at[dstv.at[b, 0]], sss[b], add=True)

        def wait_scatter(b):
            pltpu.make_async_copy(rows.at[b], acc.at[dstv.at[b, 0]], sss[b]).wait()

        zvec = jnp.zeros((L,), jnp.float32)

        @pl.loop(0, ZROWS)
        def _(r):
            for j in range(F // L):
                rows[0, r, pl.ds(j * L, L)] = zvec

        @pl.loop(0, ROWS_PER_TILE // ZROWS)
        def _(i):
            pltpu.sync_copy(rows.at[0],
                            acc.at[pl.ds(sid * ROWS_PER_TILE + i * ZROWS, ZROWS)])

        plsc.subcore_barrier()

        def scale(b):
            @pl.loop(0, CHUNK // L)
            def _(g):
                vv = plsc.bitcast(srcv[b, pl.ds(CHUNK + g * L, L)], jnp.float32)
                for l in range(L):
                    v = jnp.take(vv, jnp.full((L,), l, jnp.int32))
                    e = g * L + l
                    for j in range(F // L):
                        sl = (b, e, pl.ds(j * L, L))
                        rows[sl] = rows[sl] * v

        def half(ti, b, first=False):
            nb = 1 - b
            wait_srcval_fetch(nb)
            if not first:
                wait_scatter(nb)
            issue_dst_fetch(ti + 1, nb)
            issue_gather(nb)
            wait_gather(b)
            scale(b)
            wait_dst_fetch(b)
            issue_scatter(b)
            issue_srcval_fetch(ti + 2, b)

        issue_srcval_fetch(0, 0)
        issue_srcval_fetch(1, 1)
        wait_srcval_fetch(0)
        issue_dst_fetch(0, 0)
        issue_gather(0)

        half(0, 0, first=True)

        @pl.loop(0, (CPT - 2) // 2)
        def _(t):
            ti = 1 + 2 * t
            half(ti, 1)
            half(ti + 1, 0)

        half(CPT - 1, 1)

        wait_scatter(1)
        wait_gather(0)
        wait_dst_fetch(0)
        wait_srcval_fetch(1)

        plsc.subcore_barrier()

        @pl.loop(0, ROWS_PER_TILE // ZROWS)
        def _(i):
            off = sid * ROWS_PER_TILE + i * ZROWS

            @pl.when(cid == 0)
            def _():
                pltpu.sync_copy(acc.at[pl.ds(off, ZROWS)],
                                outa_hbm.at[pl.ds(off, ZROWS)])

            @pl.when(cid == 1)
            def _():
                pltpu.sync_copy(acc.at[pl.ds(off, ZROWS)],
                                outb_hbm.at[pl.ds(off, ZROWS)])

    return k(support, srcval, dst)


_BM = 1000


def _dot(a, b):
    return lax.dot_general(a, b, (((1,), (0,)), ((), ())),
                           precision=lax.Precision.HIGHEST,
                           preferred_element_type=jnp.float32)


def _tc_matmul(x, W):
    def body(x_ref, w_ref, o_ref):
        o_ref[...] = _dot(x_ref[...], w_ref[...])

    return pl.pallas_call(
        body,
        grid=(N // _BM,),
        in_specs=[pl.BlockSpec((_BM, F), lambda i: (i, 0)),
                  pl.BlockSpec((F, F), lambda i: (0, 0))],
        out_specs=pl.BlockSpec((_BM, F), lambda i: (i, 0)),
        out_shape=jax.ShapeDtypeStruct((N, F), jnp.float32),
    )(x, W)


def _tc_relu_matmul(pa, pb, b, W):
    def body(p0_ref, p1_ref, b_ref, w_ref, o_ref):
        h = jax.nn.relu(p0_ref[...] + p1_ref[...] + b_ref[...])
        o_ref[...] = _dot(h, w_ref[...])

    return pl.pallas_call(
        body,
        grid=(N // _BM,),
        in_specs=[pl.BlockSpec((_BM, F), lambda i: (i, 0)),
                  pl.BlockSpec((_BM, F), lambda i: (i, 0)),
                  pl.BlockSpec((1, F), lambda i: (0, 0)),
                  pl.BlockSpec((F, F), lambda i: (0, 0))],
        out_specs=pl.BlockSpec((_BM, F), lambda i: (i, 0)),
        out_shape=jax.ShapeDtypeStruct((N, F), jnp.float32),
    )(pa, pb, b.reshape(1, F), W)


def _tc_relu_bias(pa, pb, b):
    def body(p0_ref, p1_ref, b_ref, o_ref):
        o_ref[...] = jax.nn.relu(p0_ref[...] + p1_ref[...] + b_ref[...])

    return pl.pallas_call(
        body,
        grid=(N // _BM,),
        in_specs=[pl.BlockSpec((_BM, F), lambda i: (i, 0)),
                  pl.BlockSpec((_BM, F), lambda i: (i, 0)),
                  pl.BlockSpec((1, F), lambda i: (0, 0))],
        out_specs=pl.BlockSpec((_BM, F), lambda i: (i, 0)),
        out_shape=jax.ShapeDtypeStruct((N, F), jnp.float32),
    )(pa, pb, b.reshape(1, F))


def kernel(x, adj_indices, adj_values, W1, b1, W2, b2):
    dst = adj_indices[0]
    src = adj_indices[1]
    pad = EARR - E
    pidx = jnp.arange(pad, dtype=jnp.int32) % N
    src_p = jnp.concatenate([src, pidx])
    dst_p = jnp.concatenate([dst, pidx])
    vals_p = jnp.concatenate([adj_values, jnp.zeros((pad,), jnp.float32)])
    srcval = jnp.concatenate(
        [src_p.reshape(NARR, CHUNK),
         jax.lax.bitcast_convert_type(vals_p, jnp.int32).reshape(NARR, CHUNK)],
        axis=1).reshape(-1)
    s1 = _tc_matmul(x, W1)
    p1a, p1b = _spmm_sc(s1, srcval, dst_p)
    s2 = _tc_relu_matmul(p1a, p1b, b1, W2)
    p2a, p2b = _spmm_sc(s2, srcval, dst_p)
    return _tc_relu_bias(p2a, p2b, b2)
